# baseline (speedup 1.0000x reference)
import functools

import jax
import jax.numpy as jnp
from jax import lax
from jax.experimental import pallas as pl
from jax.experimental.pallas import tpu as pltpu
from jax.experimental.pallas import tpu_sc as plsc

B = 2
N = 16384
M = 2048
S = 32
L = 16
NC = 2
NS = 16
NW = NC * NS
QPW = (B * M) // NW
WPB = NW // B
CAP = 48
BUFN = 80
NCHUNK = N // L
GRP = 2

_INF = float("inf")


def _merge16(ad, ai, bd, bi):
    rbd = lax.rev(bd, (0,))
    rbi = lax.rev(bi, (0,))
    sel = ad <= rbd
    lod = jnp.where(sel, ad, rbd)
    loi = jnp.where(sel, ai, rbi)
    hid = jnp.where(sel, rbd, ad)
    hii = jnp.where(sel, rbi, ai)
    lod, loi = plsc.sort_key_val(lod, loi)
    hid, hii = plsc.sort_key_val(hid, hii)
    return (lod, loi), (hid, hii)


def _merge32_low(x0d, x0i, x1d, x1i, y0d, y0i, y1d, y1i):
    ry1d = lax.rev(y1d, (0,))
    ry1i = lax.rev(y1i, (0,))
    ry0d = lax.rev(y0d, (0,))
    ry0i = lax.rev(y0i, (0,))
    s0 = x0d <= ry1d
    lo0d = jnp.where(s0, x0d, ry1d)
    lo0i = jnp.where(s0, x0i, ry1i)
    s1 = x1d <= ry0d
    lo1d = jnp.where(s1, x1d, ry0d)
    lo1i = jnp.where(s1, x1i, ry0i)
    s2 = lo0d <= lo1d
    m0d = jnp.where(s2, lo0d, lo1d)
    m0i = jnp.where(s2, lo0i, lo1i)
    m1d = jnp.where(s2, lo1d, lo0d)
    m1i = jnp.where(s2, lo1i, lo0i)
    m0d, m0i = plsc.sort_key_val(m0d, m0i)
    m1d, m1i = plsc.sort_key_val(m1d, m1i)
    return m0d, m0i, m1d, m1i


def _rne_bf16(x):
    b = plsc.bitcast(x, jnp.uint32)
    one = jnp.uint32(1)
    r = (b + jnp.uint32(0x7FFF) + ((b >> jnp.uint32(16)) & one)) \
        & jnp.uint32(0xFFFF0000)
    return plsc.bitcast(r, jnp.float32)


def _sc_body(px_h, py_h, pz_h, qx_h, qy_h, qz_h,
             relx_o, rely_o, relz_o, idx_o,
             px_v, py_v, pz_v, psq_v, qx_v, qy_v, qz_v,
             relx_b, rely_b, relz_b, idxbuf, dbuf, ibuf, dbuf2, ibuf2):
    cid = lax.axis_index("c")
    sid = lax.axis_index("s")
    wid = cid * NS + sid
    b = wid // WPB
    qlo = (wid % WPB) * QPW

    pltpu.sync_copy(px_h.at[pl.ds(b * N, N)], px_v)
    pltpu.sync_copy(py_h.at[pl.ds(b * N, N)], py_v)
    pltpu.sync_copy(pz_h.at[pl.ds(b * N, N)], pz_v)
    qbase = b * M + qlo
    pltpu.sync_copy(qx_h.at[pl.ds(qbase, QPW)], qx_v)
    pltpu.sync_copy(qy_h.at[pl.ds(qbase, QPW)], qy_v)
    pltpu.sync_copy(qz_h.at[pl.ds(qbase, QPW)], qz_v)

    iota16 = lax.broadcasted_iota(jnp.int32, (L,), 0)

    def psq_step(i, carry):
        sl = pl.ds(i * L, L)
        px = px_v[sl]
        py = py_v[sl]
        pz = pz_v[sl]
        psq_v[sl] = (px * px + py * py) + pz * pz
        px_v[sl] = _rne_bf16(px)
        py_v[sl] = _rne_bf16(py)
        pz_v[sl] = _rne_bf16(pz)
        return carry

    lax.fori_loop(0, NCHUNK, psq_step, 0)

    def compact(count, dbuf, ibuf):
        sorted_pairs = []
        for j in range(4):
            dj = dbuf[pl.ds(j * L, L)]
            ij = ibuf[pl.ds(j * L, L)]
            valid = (iota16 + j * L) < count
            dj = jnp.where(valid, dj, _INF)
            dj, ij = plsc.sort_key_val(dj, ij)
            sorted_pairs.append((dj, ij))
        (a0, b0) = _merge16(*sorted_pairs[0], *sorted_pairs[1])
        (a1, b1) = _merge16(*sorted_pairs[2], *sorted_pairs[3])
        m0d, m0i, m1d, m1i = _merge32_low(
            a0[0], a0[1], b0[0], b0[1], a1[0], a1[1], b1[0], b1[1])
        dbuf[pl.ds(0, L)] = m0d
        ibuf[pl.ds(0, L)] = m0i
        dbuf[pl.ds(L, L)] = m1d
        ibuf[pl.ds(L, L)] = m1i
        t_scalar = lax.reduce_max(m1d, (0,))
        return jnp.full((L,), t_scalar, jnp.float32)

    def qload(qi):
        qsplat = jnp.full((L,), qi, jnp.int32)
        qx = plsc.load_gather(qx_v, [qsplat])
        qy = plsc.load_gather(qy_v, [qsplat])
        qz = plsc.load_gather(qz_v, [qsplat])
        qsq = (qx * qx + qy * qy) + qz * qz
        return qsq, _rne_bf16(qx), _rne_bf16(qy), _rne_bf16(qz)

    def scan_pair(pi, carry):
        q0 = pi * 2
        qsq0, qxr0, qyr0, qzr0 = qload(q0)
        qsq1, qxr1, qyr1, qzr1 = qload(q0 + 1)

        def make_chunk_append(ds_, cbase, dbuf, ibuf):
            def chunk_append(tc3, j):
                t3, c3 = tc3
                mask = ds_[j] < t3
                cnt_vec = plsc.all_reduce_population_count(mask)
                cnt = lax.squeeze(lax.slice(cnt_vec, (0,), (1,)), (0,))

                def has_cand(tc4):
                    t4, c4 = tc4
                    plsc.store_compressed(dbuf.at[pl.ds(c4, L)],
                                          ds_[j], mask=mask)
                    plsc.store_compressed(
                        ibuf.at[pl.ds(c4, L)],
                        iota16 + (cbase + j) * L, mask=mask)
                    c4 = c4 + cnt

                    def do_compact(tc5):
                        _, c5 = tc5
                        return compact(c5, dbuf, ibuf), jnp.int32(S)

                    return lax.cond(c4 > CAP, do_compact,
                                    lambda x: x, (t4, c4))

                return lax.cond(cnt > 0, has_cand, lambda x: x, tc3)

            return chunk_append

        def group_step(g, tc):
            t0, c0, t1, c1 = tc
            cbase = g * GRP
            ds0, ds1 = [], []
            gmin0 = gmin1 = None
            for j in range(GRP):
                sl = pl.ds((cbase + j) * L, L)
                px = px_v[sl]
                py = py_v[sl]
                pz = pz_v[sl]
                ps = psq_v[sl]
                d0 = (-2.0 * (qxr0 * px + qyr0 * py + qzr0 * pz) + qsq0) + ps
                d1 = (-2.0 * (qxr1 * px + qyr1 * py + qzr1 * pz) + qsq1) + ps
                ds0.append(d0)
                ds1.append(d1)
                gmin0 = d0 if gmin0 is None else jnp.minimum(gmin0, d0)
                gmin1 = d1 if gmin1 is None else jnp.minimum(gmin1, d1)
            hit_vec = plsc.all_reduce_population_count(
                (gmin0 < t0) | (gmin1 < t1))
            hit = lax.squeeze(lax.slice(hit_vec, (0,), (1,)), (0,))

            def slow(tc2):
                ta0, ca0, ta1, ca1 = tc2

                def sub(tq, gmin, ds_, dbufx, ibufx):
                    hv = plsc.all_reduce_population_count(gmin < tq[0])
                    h = lax.squeeze(lax.slice(hv, (0,), (1,)), (0,))

                    def do_sub(tq2):
                        app = make_chunk_append(ds_, cbase, dbufx, ibufx)
                        for j in range(GRP):
                            tq2 = app(tq2, j)
                        return tq2

                    return lax.cond(h > 0, do_sub, lambda x: x, tq)

                ta0, ca0 = sub((ta0, ca0), gmin0, ds0, dbuf, ibuf)
                ta1, ca1 = sub((ta1, ca1), gmin1, ds1, dbuf2, ibuf2)
                return ta0, ca0, ta1, ca1

            return lax.cond(hit > 0, slow, lambda x: x, tc)

        t_inf = jnp.full((L,), _INF, jnp.float32)
        _, c0, _, c1 = lax.fori_loop(
            0, NCHUNK // GRP, group_step,
            (t_inf, jnp.int32(0), t_inf, jnp.int32(0)))
        compact(c0, dbuf, ibuf)
        idxbuf[pl.ds(q0 * S, L)] = ibuf[pl.ds(0, L)]
        idxbuf[pl.ds(q0 * S + L, L)] = ibuf[pl.ds(L, L)]
        compact(c1, dbuf2, ibuf2)
        idxbuf[pl.ds((q0 + 1) * S, L)] = ibuf2[pl.ds(0, L)]
        idxbuf[pl.ds((q0 + 1) * S + L, L)] = ibuf2[pl.ds(L, L)]
        return carry

    lax.fori_loop(0, QPW // 2, scan_pair, 0)

    pltpu.sync_copy(px_h.at[pl.ds(b * N, N)], px_v)
    pltpu.sync_copy(py_h.at[pl.ds(b * N, N)], py_v)
    pltpu.sync_copy(pz_h.at[pl.ds(b * N, N)], pz_v)

    def rel_query(qi, carry):
        qsplat = jnp.full((L,), qi, jnp.int32)
        qx = plsc.load_gather(qx_v, [qsplat])
        qy = plsc.load_gather(qy_v, [qsplat])
        qz = plsc.load_gather(qz_v, [qsplat])
        for h in range(2):
            sl = pl.ds(qi * S + h * L, L)
            ii = idxbuf[sl]
            relx_b[sl] = plsc.load_gather(px_v, [ii]) - qx
            rely_b[sl] = plsc.load_gather(py_v, [ii]) - qy
            relz_b[sl] = plsc.load_gather(pz_v, [ii]) - qz
        return carry

    lax.fori_loop(0, QPW, rel_query, 0)

    obase = (b * M + qlo) * S
    pltpu.sync_copy(relx_b, relx_o.at[pl.ds(obase, QPW * S)])
    pltpu.sync_copy(rely_b, rely_o.at[pl.ds(obase, QPW * S)])
    pltpu.sync_copy(relz_b, relz_o.at[pl.ds(obase, QPW * S)])
    pltpu.sync_copy(idxbuf, idx_o.at[pl.ds(obase, QPW * S)])


_sc_knn = functools.partial(
    pl.kernel,
    out_type=[
        jax.ShapeDtypeStruct((B * M * S,), jnp.float32),
        jax.ShapeDtypeStruct((B * M * S,), jnp.float32),
        jax.ShapeDtypeStruct((B * M * S,), jnp.float32),
        jax.ShapeDtypeStruct((B * M * S,), jnp.int32),
    ],
    mesh=plsc.VectorSubcoreMesh(core_axis_name="c", subcore_axis_name="s"),
    compiler_params=pltpu.CompilerParams(needs_layout_passes=False),
    scratch_types=[
        pltpu.VMEM((N,), jnp.float32),
        pltpu.VMEM((N,), jnp.float32),
        pltpu.VMEM((N,), jnp.float32),
        pltpu.VMEM((N,), jnp.float32),
        pltpu.VMEM((QPW,), jnp.float32),
        pltpu.VMEM((QPW,), jnp.float32),
        pltpu.VMEM((QPW,), jnp.float32),
        pltpu.VMEM((QPW * S,), jnp.float32),
        pltpu.VMEM((QPW * S,), jnp.float32),
        pltpu.VMEM((QPW * S,), jnp.float32),
        pltpu.VMEM((QPW * S,), jnp.int32),
        pltpu.VMEM((BUFN,), jnp.float32),
        pltpu.VMEM((BUFN,), jnp.int32),
        pltpu.VMEM((BUFN,), jnp.float32),
        pltpu.VMEM((BUFN,), jnp.int32),
    ],
)(_sc_body)


TQA = 256


def _asm_body(rel_ref, x1t_ref, out_ref):
    rx = rel_ref[0, 0]
    ry = rel_ref[0, 1]
    rz = rel_ref[0, 2]
    dist = jnp.sqrt((rx * rx + ry * ry) + rz * rz)
    ones = jnp.ones((TQA, S), jnp.float32)
    out_ref[0, 0] = rx
    out_ref[0, 1] = ry
    out_ref[0, 2] = rz
    out_ref[0, 3] = dist
    out_ref[0, 4] = x1t_ref[0, 0][:, None] * ones
    out_ref[0, 5] = x1t_ref[0, 1][:, None] * ones
    out_ref[0, 6] = x1t_ref[0, 2][:, None] * ones


@jax.jit
def kernel(x0, x1):
    x1t = jnp.transpose(x1, (0, 2, 1))
    px = x0[:, :, 0].reshape(-1)
    py = x0[:, :, 1].reshape(-1)
    pz = x0[:, :, 2].reshape(-1)
    qx = x1[:, :, 0].reshape(-1)
    qy = x1[:, :, 1].reshape(-1)
    qz = x1[:, :, 2].reshape(-1)
    relx, rely, relz, idx = _sc_knn(px, py, pz, qx, qy, qz)
    rel = jnp.stack([relx, rely, relz]).reshape(3, B, M, S).transpose(1, 0, 2, 3)
    out = pl.pallas_call(
        _asm_body,
        grid=(B, M // TQA),
        in_specs=[
            pl.BlockSpec((1, 3, TQA, S), lambda b, m: (b, 0, m, 0)),
            pl.BlockSpec((1, 3, TQA), lambda b, m: (b, 0, m)),
        ],
        out_specs=pl.BlockSpec((1, 7, TQA, S), lambda b, m: (b, 0, m, 0)),
        out_shape=jax.ShapeDtypeStruct((B, 7, M, S), jnp.float32),
    )(rel, x1t)
    return (out, x1t, idx.reshape(B, M, S))


# speedup vs baseline: 1.1862x; 1.1862x over previous
import functools

import jax
import jax.numpy as jnp
from jax import lax
from jax.experimental import pallas as pl
from jax.experimental.pallas import tpu as pltpu
from jax.experimental.pallas import tpu_sc as plsc

B = 2
N = 16384
M = 2048
S = 32
L = 16
NC = 2
NS = 16
NW = NC * NS
QPW = (B * M) // NW
WPB = NW // B
CAP = 48
BUFN = 80
NCHUNK = N // L
GRP = 4

_INF = float("inf")


def _merge16(ad, ai, bd, bi):
    rbd = lax.rev(bd, (0,))
    rbi = lax.rev(bi, (0,))
    sel = ad <= rbd
    lod = jnp.where(sel, ad, rbd)
    loi = jnp.where(sel, ai, rbi)
    hid = jnp.where(sel, rbd, ad)
    hii = jnp.where(sel, rbi, ai)
    lod, loi = plsc.sort_key_val(lod, loi)
    hid, hii = plsc.sort_key_val(hid, hii)
    return (lod, loi), (hid, hii)


def _merge32_low(x0d, x0i, x1d, x1i, y0d, y0i, y1d, y1i):
    ry1d = lax.rev(y1d, (0,))
    ry1i = lax.rev(y1i, (0,))
    ry0d = lax.rev(y0d, (0,))
    ry0i = lax.rev(y0i, (0,))
    s0 = x0d <= ry1d
    lo0d = jnp.where(s0, x0d, ry1d)
    lo0i = jnp.where(s0, x0i, ry1i)
    s1 = x1d <= ry0d
    lo1d = jnp.where(s1, x1d, ry0d)
    lo1i = jnp.where(s1, x1i, ry0i)
    s2 = lo0d <= lo1d
    m0d = jnp.where(s2, lo0d, lo1d)
    m0i = jnp.where(s2, lo0i, lo1i)
    m1d = jnp.where(s2, lo1d, lo0d)
    m1i = jnp.where(s2, lo1i, lo0i)
    m0d, m0i = plsc.sort_key_val(m0d, m0i)
    m1d, m1i = plsc.sort_key_val(m1d, m1i)
    return m0d, m0i, m1d, m1i


def _rne_bf16(x):
    b = plsc.bitcast(x, jnp.uint32)
    one = jnp.uint32(1)
    r = (b + jnp.uint32(0x7FFF) + ((b >> jnp.uint32(16)) & one)) \
        & jnp.uint32(0xFFFF0000)
    return plsc.bitcast(r, jnp.float32)


def _sc_body(px_h, py_h, pz_h, qx_h, qy_h, qz_h,
             relx_o, rely_o, relz_o, idx_o,
             px_v, py_v, pz_v, psq_v, qx_v, qy_v, qz_v,
             relx_b, rely_b, relz_b, idxbuf, dbuf, ibuf, dbuf2, ibuf2,
             dbuf3, ibuf3, dbuf4, ibuf4):
    cid = lax.axis_index("c")
    sid = lax.axis_index("s")
    wid = cid * NS + sid
    b = wid // WPB
    qlo = (wid % WPB) * QPW

    pltpu.sync_copy(px_h.at[pl.ds(b * N, N)], px_v)
    pltpu.sync_copy(py_h.at[pl.ds(b * N, N)], py_v)
    pltpu.sync_copy(pz_h.at[pl.ds(b * N, N)], pz_v)
    qbase = b * M + qlo
    pltpu.sync_copy(qx_h.at[pl.ds(qbase, QPW)], qx_v)
    pltpu.sync_copy(qy_h.at[pl.ds(qbase, QPW)], qy_v)
    pltpu.sync_copy(qz_h.at[pl.ds(qbase, QPW)], qz_v)

    iota16 = lax.broadcasted_iota(jnp.int32, (L,), 0)

    def psq_step(i, carry):
        sl = pl.ds(i * L, L)
        px = px_v[sl]
        py = py_v[sl]
        pz = pz_v[sl]
        psq_v[sl] = (px * px + py * py) + pz * pz
        px_v[sl] = _rne_bf16(px)
        py_v[sl] = _rne_bf16(py)
        pz_v[sl] = _rne_bf16(pz)
        return carry

    lax.fori_loop(0, NCHUNK, psq_step, 0)

    def compact(count, dbuf, ibuf):
        sorted_pairs = []
        for j in range(4):
            dj = dbuf[pl.ds(j * L, L)]
            ij = ibuf[pl.ds(j * L, L)]
            valid = (iota16 + j * L) < count
            dj = jnp.where(valid, dj, _INF)
            dj, ij = plsc.sort_key_val(dj, ij)
            sorted_pairs.append((dj, ij))
        (a0, b0) = _merge16(*sorted_pairs[0], *sorted_pairs[1])
        (a1, b1) = _merge16(*sorted_pairs[2], *sorted_pairs[3])
        m0d, m0i, m1d, m1i = _merge32_low(
            a0[0], a0[1], b0[0], b0[1], a1[0], a1[1], b1[0], b1[1])
        dbuf[pl.ds(0, L)] = m0d
        ibuf[pl.ds(0, L)] = m0i
        dbuf[pl.ds(L, L)] = m1d
        ibuf[pl.ds(L, L)] = m1i
        t_scalar = lax.reduce_max(m1d, (0,))
        return jnp.full((L,), t_scalar, jnp.float32)

    def qload(qi):
        qsplat = jnp.full((L,), qi, jnp.int32)
        qx = plsc.load_gather(qx_v, [qsplat])
        qy = plsc.load_gather(qy_v, [qsplat])
        qz = plsc.load_gather(qz_v, [qsplat])
        qsq = (qx * qx + qy * qy) + qz * qz
        return qsq, _rne_bf16(qx), _rne_bf16(qy), _rne_bf16(qz)

    def scan_quad(pi, carry):
        q0 = pi * 4
        NQ = 4
        bufs = [(dbuf, ibuf), (dbuf2, ibuf2), (dbuf3, ibuf3), (dbuf4, ibuf4)]
        qs = [qload(q0 + k) for k in range(NQ)]

        def make_chunk_append(ds_, cbase, dbufx, ibufx):
            def chunk_append(tc3, j):
                t3, c3 = tc3
                mask = ds_[j] < t3
                cnt_vec = plsc.all_reduce_population_count(mask)
                cnt = lax.squeeze(lax.slice(cnt_vec, (0,), (1,)), (0,))

                def has_cand(tc4):
                    t4, c4 = tc4
                    plsc.store_compressed(dbufx.at[pl.ds(c4, L)],
                                          ds_[j], mask=mask)
                    plsc.store_compressed(
                        ibufx.at[pl.ds(c4, L)],
                        iota16 + (cbase + j) * L, mask=mask)
                    c4 = c4 + cnt

                    def do_compact(tc5):
                        _, c5 = tc5
                        return compact(c5, dbufx, ibufx), jnp.int32(S)

                    return lax.cond(c4 > CAP, do_compact,
                                    lambda x: x, (t4, c4))

                return lax.cond(cnt > 0, has_cand, lambda x: x, tc3)

            return chunk_append

        def group_step(g, tc):
            ts = list(tc[0:NQ])
            cs = list(tc[NQ:2 * NQ])
            cbase = g * GRP
            ds = [[] for _ in range(NQ)]
            gmin = [None] * NQ
            for j in range(GRP):
                sl = pl.ds((cbase + j) * L, L)
                px = px_v[sl]
                py = py_v[sl]
                pz = pz_v[sl]
                ps = psq_v[sl]
                for k in range(NQ):
                    qsq, qxr, qyr, qzr = qs[k]
                    d = (-2.0 * (qxr * px + qyr * py + qzr * pz) + qsq) + ps
                    ds[k].append(d)
                    gmin[k] = d if gmin[k] is None else jnp.minimum(gmin[k], d)
            anyhit = (gmin[0] < ts[0]) | (gmin[1] < ts[1])
            anyhit = anyhit | (gmin[2] < ts[2]) | (gmin[3] < ts[3])
            hit_vec = plsc.all_reduce_population_count(anyhit)
            hit = lax.squeeze(lax.slice(hit_vec, (0,), (1,)), (0,))

            def slow(tc2):
                ts2 = list(tc2[0:NQ])
                cs2 = list(tc2[NQ:2 * NQ])

                def sub(tq, gm, ds_, dbufx, ibufx):
                    hv = plsc.all_reduce_population_count(gm < tq[0])
                    h = lax.squeeze(lax.slice(hv, (0,), (1,)), (0,))

                    def do_sub(tq2):
                        app = make_chunk_append(ds_, cbase, dbufx, ibufx)
                        for j in range(GRP):
                            tq2 = app(tq2, j)
                        return tq2

                    return lax.cond(h > 0, do_sub, lambda x: x, tq)

                for k in range(NQ):
                    ts2[k], cs2[k] = sub((ts2[k], cs2[k]), gmin[k], ds[k],
                                         bufs[k][0], bufs[k][1])
                return tuple(ts2) + tuple(cs2)

            return lax.cond(hit > 0, slow, lambda x: x, tc)

        t_inf = jnp.full((L,), _INF, jnp.float32)
        res = lax.fori_loop(
            0, NCHUNK // GRP, group_step,
            tuple([t_inf] * NQ) + tuple([jnp.int32(0)] * NQ))
        for k in range(NQ):
            compact(res[NQ + k], bufs[k][0], bufs[k][1])
            idxbuf[pl.ds((q0 + k) * S, L)] = bufs[k][1][pl.ds(0, L)]
            idxbuf[pl.ds((q0 + k) * S + L, L)] = bufs[k][1][pl.ds(L, L)]
        return carry

    lax.fori_loop(0, QPW // 4, scan_quad, 0)

    pltpu.sync_copy(px_h.at[pl.ds(b * N, N)], px_v)
    pltpu.sync_copy(py_h.at[pl.ds(b * N, N)], py_v)
    pltpu.sync_copy(pz_h.at[pl.ds(b * N, N)], pz_v)

    def rel_query(qi, carry):
        qsplat = jnp.full((L,), qi, jnp.int32)
        qx = plsc.load_gather(qx_v, [qsplat])
        qy = plsc.load_gather(qy_v, [qsplat])
        qz = plsc.load_gather(qz_v, [qsplat])
        for h in range(2):
            sl = pl.ds(qi * S + h * L, L)
            ii = idxbuf[sl]
            relx_b[sl] = plsc.load_gather(px_v, [ii]) - qx
            rely_b[sl] = plsc.load_gather(py_v, [ii]) - qy
            relz_b[sl] = plsc.load_gather(pz_v, [ii]) - qz
        return carry

    lax.fori_loop(0, QPW, rel_query, 0)

    obase = (b * M + qlo) * S
    pltpu.sync_copy(relx_b, relx_o.at[pl.ds(obase, QPW * S)])
    pltpu.sync_copy(rely_b, rely_o.at[pl.ds(obase, QPW * S)])
    pltpu.sync_copy(relz_b, relz_o.at[pl.ds(obase, QPW * S)])
    pltpu.sync_copy(idxbuf, idx_o.at[pl.ds(obase, QPW * S)])


_sc_knn = functools.partial(
    pl.kernel,
    out_type=[
        jax.ShapeDtypeStruct((B * M * S,), jnp.float32),
        jax.ShapeDtypeStruct((B * M * S,), jnp.float32),
        jax.ShapeDtypeStruct((B * M * S,), jnp.float32),
        jax.ShapeDtypeStruct((B * M * S,), jnp.int32),
    ],
    mesh=plsc.VectorSubcoreMesh(core_axis_name="c", subcore_axis_name="s"),
    compiler_params=pltpu.CompilerParams(needs_layout_passes=False),
    scratch_types=[
        pltpu.VMEM((N,), jnp.float32),
        pltpu.VMEM((N,), jnp.float32),
        pltpu.VMEM((N,), jnp.float32),
        pltpu.VMEM((N,), jnp.float32),
        pltpu.VMEM((QPW,), jnp.float32),
        pltpu.VMEM((QPW,), jnp.float32),
        pltpu.VMEM((QPW,), jnp.float32),
        pltpu.VMEM((QPW * S,), jnp.float32),
        pltpu.VMEM((QPW * S,), jnp.float32),
        pltpu.VMEM((QPW * S,), jnp.float32),
        pltpu.VMEM((QPW * S,), jnp.int32),
        pltpu.VMEM((BUFN,), jnp.float32),
        pltpu.VMEM((BUFN,), jnp.int32),
        pltpu.VMEM((BUFN,), jnp.float32),
        pltpu.VMEM((BUFN,), jnp.int32),
        pltpu.VMEM((BUFN,), jnp.float32),
        pltpu.VMEM((BUFN,), jnp.int32),
        pltpu.VMEM((BUFN,), jnp.float32),
        pltpu.VMEM((BUFN,), jnp.int32),
    ],
)(_sc_body)


TQA = 256


def _asm_body(rel_ref, x1t_ref, out_ref):
    rx = rel_ref[0, 0]
    ry = rel_ref[0, 1]
    rz = rel_ref[0, 2]
    dist = jnp.sqrt((rx * rx + ry * ry) + rz * rz)
    ones = jnp.ones((TQA, S), jnp.float32)
    out_ref[0, 0] = rx
    out_ref[0, 1] = ry
    out_ref[0, 2] = rz
    out_ref[0, 3] = dist
    out_ref[0, 4] = x1t_ref[0, 0][:, None] * ones
    out_ref[0, 5] = x1t_ref[0, 1][:, None] * ones
    out_ref[0, 6] = x1t_ref[0, 2][:, None] * ones


@jax.jit
def kernel(x0, x1):
    x1t = jnp.transpose(x1, (0, 2, 1))
    px = x0[:, :, 0].reshape(-1)
    py = x0[:, :, 1].reshape(-1)
    pz = x0[:, :, 2].reshape(-1)
    qx = x1[:, :, 0].reshape(-1)
    qy = x1[:, :, 1].reshape(-1)
    qz = x1[:, :, 2].reshape(-1)
    relx, rely, relz, idx = _sc_knn(px, py, pz, qx, qy, qz)
    rel = jnp.stack([relx, rely, relz]).reshape(3, B, M, S).transpose(1, 0, 2, 3)
    out = pl.pallas_call(
        _asm_body,
        grid=(B, M // TQA),
        in_specs=[
            pl.BlockSpec((1, 3, TQA, S), lambda b, m: (b, 0, m, 0)),
            pl.BlockSpec((1, 3, TQA), lambda b, m: (b, 0, m)),
        ],
        out_specs=pl.BlockSpec((1, 7, TQA, S), lambda b, m: (b, 0, m, 0)),
        out_shape=jax.ShapeDtypeStruct((B, 7, M, S), jnp.float32),
    )(rel, x1t)
    return (out, x1t, idx.reshape(B, M, S))
